# baseline (speedup 1.0000x reference)
import functools

import jax
import jax.numpy as jnp
from jax import lax
from jax.experimental import pallas as pl
from jax.experimental.pallas import tpu as pltpu
from jax.experimental.pallas import tpu_sc as plsc

B = 16384
D = 32
NROWS = 1000000
NC = 2
NS = 16
NW = NC * NS
N_TC = 4096
N_SC = B - N_TC
B_PER_W = N_SC // NW
TC_UNROLL = 4
WAVE = 8
NWAVE = B_PER_W // WAVE
NBAND = 4
CLEAN = (NROWS // 128) * 128
NTAIL = NROWS - CLEAN
MAXTC = CLEAN // 128 - 1


def _tile_block_gather():
    mesh = plsc.VectorSubcoreMesh(core_axis_name="c", subcore_axis_name="s")

    @functools.partial(
        pl.kernel,
        mesh=mesh,
        out_type=jax.ShapeDtypeStruct((D, N_SC), jnp.float32),
        scratch_types=[
            pltpu.VMEM((B_PER_W,), jnp.int32),
            pltpu.VMEM((2 * WAVE, D, 128), jnp.float32),
            pltpu.VMEM((D, B_PER_W), jnp.float32),
            pltpu.VMEM((NTAIL * D,), jnp.float32),
            pltpu.SemaphoreType.DMA,
            pltpu.SemaphoreType.DMA,
        ],
        compiler_params=pltpu.CompilerParams(
            use_tc_tiling_on_sc=True, needs_layout_passes=False
        ),
    )
    def k(table_hbm, idx_hbm, tail_hbm, out_hbm, idx_v, staged, slab, tailv,
          sem0, sem1):
        wid = lax.axis_index("s") * NC + lax.axis_index("c")
        base = wid * B_PER_W
        pltpu.sync_copy(idx_hbm.at[pl.ds(base, B_PER_W)], idx_v)
        pltpu.sync_copy(tail_hbm, tailv)

        iota = lax.iota(jnp.int32, 16)

        NB = B_PER_W // 16

        def batch_vec(b):
            return idx_v[pl.ds(b * 16, 16)]

        def wave_scalars(ivec, h):
            return [ivec[h * WAVE + t] for t in range(WAVE)]

        table_3d = table_hbm.reshape(NBAND, 8, 1000000)

        def fire(ivec, h, p, sem):
            for t, i in enumerate(wave_scalars(ivec, h)):
                tc = jnp.minimum(i >> 7, MAXTC)
                tcoff = pl.multiple_of(tc << 7, 128)
                pltpu.async_copy(
                    table_3d.at[:, :, pl.ds(tcoff, 128)],
                    staged.at[p * WAVE + t].reshape(NBAND, 8, 128),
                    sem,
                )

        def drain(sem):
            for _ in range(WAVE):
                pltpu.make_async_copy(
                    table_3d.at[:, :, pl.ds(0, 128)],
                    staged.at[0].reshape(NBAND, 8, 128),
                    sem,
                ).wait()

        def extract(ivec, h, p, col0):
            for t, i in enumerate(wave_scalars(ivec, h)):
                slot = jnp.full((16,), p * WAVE + t, jnp.int32)
                lane = jnp.full((16,), i & 127, jnp.int32)
                lo = plsc.load_gather(staged, [slot, iota, lane])
                hi = plsc.load_gather(staged, [slot, iota + 16, lane])
                tpos = jnp.clip(i - CLEAN, 0, NTAIL - 1) * D + iota
                tlo = plsc.load_gather(tailv, [tpos])
                thi = plsc.load_gather(tailv, [tpos + 16])
                is_tail = jnp.full((16,), i, jnp.int32) >= CLEAN
                lo = jnp.where(is_tail, tlo, lo)
                hi = jnp.where(is_tail, thi, hi)
                colv = jnp.full((16,), col0 + t, jnp.int32)
                plsc.store_scatter(slab, [iota, colv], lo)
                plsc.store_scatter(slab, [iota + 16, colv], hi)

        fire(batch_vec(0), 0, 0, sem0)

        def body(b, _):
            ivec = batch_vec(b)
            ivec_next = idx_v[pl.ds(jnp.minimum(b + 1, NB - 1) * 16, 16)]

            drain(sem0)
            fire(ivec, 1, 1, sem1)
            extract(ivec, 0, 0, b * 16)

            drain(sem1)

            @pl.when(b + 1 < NB)
            def _():
                fire(ivec_next, 0, 0, sem0)

            extract(ivec, 1, 1, b * 16 + WAVE)
            return ()

        lax.fori_loop(0, NB, body, (), unroll=False)
        pltpu.sync_copy(slab, out_hbm.at[:, pl.ds(base, B_PER_W)])

    return k


def _tc_gather():
    def body(idx_ref, t_refs, tail_ref, out_ref):
        step = pl.program_id(0)
        lane_iota = lax.broadcasted_iota(jnp.int32, (128, 1), 0)
        trow_iota = lax.broadcasted_iota(jnp.int32, (NTAIL, 1), 0)
        col_iota = lax.broadcasted_iota(jnp.int32, (D, 128), 1)
        merged = out_ref[...]
        for u in range(TC_UNROLL):
            iv = idx_ref[step * TC_UNROLL + u]
            onehot = (lane_iota == (iv & 127)).astype(jnp.float32)
            x = jnp.dot(t_refs[u][...], onehot,
                        preferred_element_type=jnp.float32,
                        precision=lax.Precision.HIGHEST)
            t = jnp.clip(iv - CLEAN, 0, NTAIL - 1)
            toh = (trow_iota == t).astype(jnp.float32)
            xt = jnp.dot(tail_ref[...], toh,
                         preferred_element_type=jnp.float32,
                         precision=lax.Precision.HIGHEST)
            x = jnp.where(iv >= CLEAN, xt, x)
            col = (step * TC_UNROLL + u) % 128
            merged = jnp.where(col_iota == col,
                               jnp.broadcast_to(x, (D, 128)), merged)
        out_ref[...] = merged

    def body_flat(idx_ref, t0, t1, t2, t3, tail_ref, out_ref):
        return body(idx_ref, (t0, t1, t2, t3), tail_ref, out_ref)

    def table_spec(u):
        return pl.BlockSpec(
            (D, 128),
            lambda i, idx_ref, u=u: (
                0,
                jnp.minimum(idx_ref[i * TC_UNROLL + u] >> 7, MAXTC),
            ),
        )

    grid_spec = pltpu.PrefetchScalarGridSpec(
        num_scalar_prefetch=1,
        grid=(N_TC // TC_UNROLL,),
        in_specs=[
            *[table_spec(u) for u in range(TC_UNROLL)],
            pl.BlockSpec((D, NTAIL), lambda i, idx_ref: (0, 0)),
        ],
        out_specs=pl.BlockSpec(
            (D, 128), lambda i, idx_ref: (0, (i * TC_UNROLL) // 128)
        ),
    )
    return pl.pallas_call(
        body_flat,
        grid_spec=grid_spec,
        out_shape=jax.ShapeDtypeStruct((D, N_TC), jnp.float32),
    )


def kernel(dataset, indices):
    table_t = dataset.T
    idx = indices.astype(jnp.int32)
    tail2d = lax.slice(dataset, (CLEAN, 0), (NROWS, D))
    tail = tail2d.reshape(-1)
    out_sc = _tile_block_gather()(
        table_t, lax.slice(idx, (0,), (N_SC,)), tail
    )
    out_tc = _tc_gather()(
        lax.slice(idx, (N_SC,), (B,)), table_t, table_t, table_t, table_t,
        tail2d.T,
    )
    out_t = jnp.concatenate([out_sc, out_tc], axis=1)
    return out_t.T


# speedup vs baseline: 4.8212x; 4.8212x over previous
import functools

import jax
import jax.numpy as jnp
from jax import lax
from jax.experimental import pallas as pl
from jax.experimental.pallas import tpu as pltpu
from jax.experimental.pallas import tpu_sc as plsc

B = 16384
D = 32
NROWS = 1000000
NC = 2
NS = 16
NW = NC * NS
B_PER_W = B // NW
WAVE = 8
NWAVE = B_PER_W // WAVE
NBAND = 4
CLEAN = (NROWS // 128) * 128
NTAIL = NROWS - CLEAN
MAXTC = CLEAN // 128 - 1


def _tile_block_gather():
    mesh = plsc.VectorSubcoreMesh(core_axis_name="c", subcore_axis_name="s")

    @functools.partial(
        pl.kernel,
        mesh=mesh,
        out_type=jax.ShapeDtypeStruct((D, B), jnp.float32),
        scratch_types=[
            pltpu.VMEM((B_PER_W,), jnp.int32),
            pltpu.VMEM((2 * WAVE, D, 128), jnp.float32),
            pltpu.VMEM((D, B_PER_W), jnp.float32),
            pltpu.VMEM((NTAIL * D,), jnp.float32),
            pltpu.SemaphoreType.DMA,
            pltpu.SemaphoreType.DMA,
        ],
        compiler_params=pltpu.CompilerParams(
            use_tc_tiling_on_sc=True, needs_layout_passes=False
        ),
    )
    def k(table_hbm, idx_hbm, tail_hbm, out_hbm, idx_v, staged, slab, tailv,
          sem0, sem1):
        wid = lax.axis_index("s") * NC + lax.axis_index("c")
        base = wid * B_PER_W
        pltpu.sync_copy(idx_hbm.at[pl.ds(base, B_PER_W)], idx_v)
        pltpu.sync_copy(tail_hbm, tailv)

        iota = lax.iota(jnp.int32, 16)

        NB = B_PER_W // 16

        def batch_vec(b):
            return idx_v[pl.ds(b * 16, 16)]

        def wave_scalars(ivec, h):
            return [ivec[h * WAVE + t] for t in range(WAVE)]

        table_3d = table_hbm.reshape(NBAND, 8, 1000000)

        def fire(ivec, h, p, sem):
            for t, i in enumerate(wave_scalars(ivec, h)):
                tc = jnp.minimum(i >> 7, MAXTC)
                tcoff = pl.multiple_of(tc << 7, 128)
                pltpu.async_copy(
                    table_3d.at[:, :, pl.ds(tcoff, 128)],
                    staged.at[p * WAVE + t].reshape(NBAND, 8, 128),
                    sem,
                )

        def drain(sem):
            for _ in range(WAVE):
                pltpu.make_async_copy(
                    table_3d.at[:, :, pl.ds(0, 128)],
                    staged.at[0].reshape(NBAND, 8, 128),
                    sem,
                ).wait()

        def extract(ivec, h, p, col0):
            for t, i in enumerate(wave_scalars(ivec, h)):
                slot = jnp.full((16,), p * WAVE + t, jnp.int32)
                lane = jnp.full((16,), i & 127, jnp.int32)
                lo = plsc.load_gather(staged, [slot, iota, lane])
                hi = plsc.load_gather(staged, [slot, iota + 16, lane])
                tpos = jnp.clip(i - CLEAN, 0, NTAIL - 1) * D + iota
                tlo = plsc.load_gather(tailv, [tpos])
                thi = plsc.load_gather(tailv, [tpos + 16])
                is_tail = jnp.full((16,), i, jnp.int32) >= CLEAN
                lo = jnp.where(is_tail, tlo, lo)
                hi = jnp.where(is_tail, thi, hi)
                colv = jnp.full((16,), col0 + t, jnp.int32)
                plsc.store_scatter(slab, [iota, colv], lo)
                plsc.store_scatter(slab, [iota + 16, colv], hi)

        fire(batch_vec(0), 0, 0, sem0)

        def body(b, _):
            ivec = batch_vec(b)
            ivec_next = idx_v[pl.ds(jnp.minimum(b + 1, NB - 1) * 16, 16)]

            drain(sem0)
            fire(ivec, 1, 1, sem1)
            extract(ivec, 0, 0, b * 16)

            drain(sem1)

            @pl.when(b + 1 < NB)
            def _():
                fire(ivec_next, 0, 0, sem0)

            extract(ivec, 1, 1, b * 16 + WAVE)
            return ()

        lax.fori_loop(0, NB, body, (), unroll=False)
        pltpu.sync_copy(slab, out_hbm.at[:, pl.ds(base, B_PER_W)])

    return k


def kernel(dataset, indices):
    table_t = dataset.T
    idx = indices.astype(jnp.int32)
    tail = lax.slice(dataset, (CLEAN, 0), (NROWS, D)).reshape(-1)
    out_t = _tile_block_gather()(table_t, idx, tail)
    return out_t.T
